# Initial kernel scaffold; baseline (speedup 1.0000x reference)
import jax
import jax.numpy as jnp
from jax.experimental import pallas as pl


def kernel(x):
    raise NotImplementedError("write your pallas kernel here")


# speedup vs baseline: 15.6876x; 15.6876x over previous
import jax
import jax.numpy as jnp
from jax.experimental import pallas as pl
from jax.experimental.pallas import tpu as pltpu

_TOPK = 1024
_ROWS = 64
_COLS = 32768
_BLOCK_ROWS = 8


def _body(x_ref, o_ref):
    x = x_ref[...]
    u = jax.lax.bitcast_convert_type(x, jnp.uint32)
    sign = u >> jnp.uint32(31)
    key = u ^ (jnp.uint32(0x80000000) + sign * jnp.uint32(0x7FFFFFFF))

    def step(i, p):
        bit = jnp.uint32(31) - i.astype(jnp.uint32)
        cand = p | (jnp.uint32(1) << bit)
        cnt = jnp.sum((key >= cand).astype(jnp.int32), axis=1, keepdims=True)
        return jnp.where(cnt >= _TOPK, cand, p)

    p0 = jnp.zeros((x.shape[0], 1), jnp.uint32)
    thresh = jax.lax.fori_loop(0, 32, step, p0)
    o_ref[...] = jnp.where(key >= thresh, x, jnp.float32(0.0))


def kernel(x):
    return pl.pallas_call(
        _body,
        out_shape=jax.ShapeDtypeStruct((_ROWS, _COLS), jnp.float32),
        grid=(_ROWS // _BLOCK_ROWS,),
        in_specs=[pl.BlockSpec((_BLOCK_ROWS, _COLS), lambda i: (i, 0))],
        out_specs=pl.BlockSpec((_BLOCK_ROWS, _COLS), lambda i: (i, 0)),
    )(x)
